# baseline (speedup 1.0000x reference)
import functools

import jax
import jax.numpy as jnp
from jax import lax
from jax.experimental import pallas as pl
from jax.experimental.pallas import tpu as pltpu
from jax.experimental.pallas import tpu_sc as plsc

_N = 524288
_D = 16
_TABLE = 128 ** 3
_MASK = _TABLE - 1

_NC = 2
_NS = 16
_NW = _NC * _NS
_BPW = _N // _NW

_C = 1024
_NCH = _BPW // _C
_G = _C // 128
_HI = _C // 16

_P1 = -1640531535
_P2 = 805459861


def _quant(v):
    f = (v + 1.0) * 64.0
    f = jnp.minimum(jnp.maximum(f, 0.0), 127.0)
    return f.astype(jnp.int32)


def _hash_kernel(xyz_hbm, table_hbm, out_hbm, xyz_v, idx_v, rows_v, sem):
    wid = lax.axis_index("s") * _NC + lax.axis_index("c")
    base = wid * _BPW
    lanes3 = lax.iota(jnp.int32, 16) * 3

    def chunk_body(c, carry):
        cb = base + c * _C
        pltpu.sync_copy(xyz_hbm.at[pl.ds(cb * 3, _C * 3)], xyz_v)
        for j in range(_HI):
            ix = lanes3 + (j * 48)
            x = plsc.load_gather(xyz_v, [ix])
            y = plsc.load_gather(xyz_v, [ix + 1])
            z = plsc.load_gather(xyz_v, [ix + 2])
            h = _quant(x) ^ (_quant(y) * _P1) ^ (_quant(z) * _P2)
            idx_v[j // 8, pl.ds((j % 8) * 16, 16)] = h & _MASK
        copies = [
            pltpu.async_copy(
                table_hbm.at[idx_v.at[g]],
                rows_v.at[pl.ds(g * 128, 128)],
                sem,
            )
            for g in range(_G)
        ]
        for cp in copies:
            cp.wait()
        pltpu.sync_copy(rows_v, out_hbm.at[pl.ds(cb, _C)])
        return carry

    lax.fori_loop(0, _NCH, chunk_body, 0)


@jax.jit
def kernel(xyz, table):
    mesh = plsc.VectorSubcoreMesh(core_axis_name="c", subcore_axis_name="s")
    run = functools.partial(
        pl.kernel,
        mesh=mesh,
        out_type=jax.ShapeDtypeStruct((_N, _D), jnp.float32),
        scratch_types=[
            pltpu.VMEM((_C * 3,), jnp.float32),
            pltpu.VMEM((_G, 128), jnp.int32),
            pltpu.VMEM((_C, _D), jnp.float32),
            pltpu.SemaphoreType.DMA,
        ],
        compiler_params=pltpu.CompilerParams(
            needs_layout_passes=False, use_tc_tiling_on_sc=False
        ),
    )(_hash_kernel)
    return run(xyz.reshape(_N * 3), table)


# speedup vs baseline: 1.3794x; 1.3794x over previous
import functools

import jax
import jax.numpy as jnp
from jax import lax
from jax.experimental import pallas as pl
from jax.experimental.pallas import tpu as pltpu
from jax.experimental.pallas import tpu_sc as plsc

_N = 524288
_D = 16
_TABLE = 128 ** 3
_MASK = _TABLE - 1

_NC = 2
_NS = 16
_NW = _NC * _NS
_BPW = _N // _NW

_C = 1024
_NCH = _BPW // _C
_G = _C // 128
_HI = _C // 16

_CT = _TABLE // 128
_CT_PW = _CT // _NW
_OCT_PW = _BPW // 128

_P1 = -1640531535
_P2 = 805459861


def _worker_id():
    return lax.axis_index("s") * _NC + lax.axis_index("c")


def _quant(v):
    f = (v + 1.0) * 64.0
    f = jnp.minimum(jnp.maximum(f, 0.0), 127.0)
    return f.astype(jnp.int32)


def _table_transpose_kernel(tt_hbm, lin_hbm, lo_v, hi_v, dst_v):
    base_ct = _worker_id() * _CT_PW
    lanes16 = lax.iota(jnp.int32, 16) * _D

    def ct_body(i, carry):
        c = base_ct + i
        pltpu.sync_copy(tt_hbm.at[pl.ds(0, 8), pl.ds(c * 128, 128)], lo_v)
        pltpu.sync_copy(tt_hbm.at[pl.ds(8, 8), pl.ds(c * 128, 128)], hi_v)
        for e in range(8):
            for k in range(8):
                off = k * 256 + e
                plsc.store_scatter(dst_v, [lanes16 + off],
                                   lo_v[e, pl.ds(k * 16, 16)])
                plsc.store_scatter(dst_v, [lanes16 + (off + 8)],
                                   hi_v[e, pl.ds(k * 16, 16)])
        pltpu.sync_copy(dst_v, lin_hbm.at[pl.ds(c * 2048, 2048)])
        return carry

    lax.fori_loop(0, _CT_PW, ct_body, 0)


def _gather_kernel(x_hbm, y_hbm, z_hbm, tbl_hbm, out_hbm,
                   x_v, y_v, z_v, idx_v, rows_v, sem):
    base = _worker_id() * _BPW

    def chunk_body(c, carry):
        cb = base + c * _C
        pltpu.sync_copy(x_hbm.at[pl.ds(cb, _C)], x_v)
        pltpu.sync_copy(y_hbm.at[pl.ds(cb, _C)], y_v)
        pltpu.sync_copy(z_hbm.at[pl.ds(cb, _C)], z_v)
        for j in range(_HI):
            s = pl.ds(j * 16, 16)
            h = (_quant(x_v[s])
                 ^ (_quant(y_v[s]) * _P1)
                 ^ (_quant(z_v[s]) * _P2))
            idx_v[j // 8, pl.ds((j % 8) * 16, 16)] = h & _MASK
        copies = [
            pltpu.async_copy(
                tbl_hbm.at[idx_v.at[g]],
                rows_v.at[pl.ds(g * 128, 128)],
                sem,
            )
            for g in range(_G)
        ]
        for cp in copies:
            cp.wait()
        pltpu.sync_copy(rows_v, out_hbm.at[pl.ds(cb, _C)])
        return carry

    lax.fori_loop(0, _NCH, chunk_body, 0)


def _out_transpose_kernel(lin_hbm, ot_hbm, src_v, lo_v, hi_v):
    base_p = _worker_id() * _BPW
    lanes16 = lax.iota(jnp.int32, 16) * _D

    def ct_body(i, carry):
        p0 = base_p + i * 128
        pltpu.sync_copy(lin_hbm.at[pl.ds(p0 * _D, 2048)], src_v)
        for e in range(8):
            for k in range(8):
                off = k * 256 + e
                lo_v[e, pl.ds(k * 16, 16)] = plsc.load_gather(
                    src_v, [lanes16 + off])
                hi_v[e, pl.ds(k * 16, 16)] = plsc.load_gather(
                    src_v, [lanes16 + (off + 8)])
        pltpu.sync_copy(lo_v, ot_hbm.at[pl.ds(0, 8), pl.ds(p0, 128)])
        pltpu.sync_copy(hi_v, ot_hbm.at[pl.ds(8, 8), pl.ds(p0, 128)])
        return carry

    lax.fori_loop(0, _OCT_PW, ct_body, 0)


@jax.jit
def kernel(xyz, table):
    mesh = plsc.VectorSubcoreMesh(core_axis_name="c", subcore_axis_name="s")
    tiled_params = pltpu.CompilerParams(
        needs_layout_passes=False, use_tc_tiling_on_sc=True
    )
    linear_params = pltpu.CompilerParams(
        needs_layout_passes=False, use_tc_tiling_on_sc=False
    )

    transpose_table = functools.partial(
        pl.kernel,
        mesh=mesh,
        out_type=jax.ShapeDtypeStruct((_TABLE * _D,), jnp.float32),
        scratch_types=[
            pltpu.VMEM((8, 128), jnp.float32),
            pltpu.VMEM((8, 128), jnp.float32),
            pltpu.VMEM((2048,), jnp.float32),
        ],
        compiler_params=tiled_params,
    )(_table_transpose_kernel)

    gather = functools.partial(
        pl.kernel,
        mesh=mesh,
        out_type=jax.ShapeDtypeStruct((_N, _D), jnp.float32),
        scratch_types=[
            pltpu.VMEM((_C,), jnp.float32),
            pltpu.VMEM((_C,), jnp.float32),
            pltpu.VMEM((_C,), jnp.float32),
            pltpu.VMEM((_G, 128), jnp.int32),
            pltpu.VMEM((_C, _D), jnp.float32),
            pltpu.SemaphoreType.DMA,
        ],
        compiler_params=linear_params,
    )(_gather_kernel)

    transpose_out = functools.partial(
        pl.kernel,
        mesh=mesh,
        out_type=jax.ShapeDtypeStruct((_D, _N), jnp.float32),
        scratch_types=[
            pltpu.VMEM((2048,), jnp.float32),
            pltpu.VMEM((8, 128), jnp.float32),
            pltpu.VMEM((8, 128), jnp.float32),
        ],
        compiler_params=tiled_params,
    )(_out_transpose_kernel)

    x, y, z = xyz[:, 0], xyz[:, 1], xyz[:, 2]
    tbl_lin = transpose_table(table.T)
    out_lin = gather(x, y, z, tbl_lin.reshape(_TABLE, _D))
    out_t = transpose_out(out_lin.reshape(_N * _D))
    return out_t.T


# speedup vs baseline: 2.9176x; 2.1151x over previous
import functools

import jax
import jax.numpy as jnp
from jax import lax
from jax.experimental import pallas as pl
from jax.experimental.pallas import tpu as pltpu
from jax.experimental.pallas import tpu_sc as plsc

_N = 524288
_D = 16
_TABLE = 128 ** 3
_MASK = _TABLE - 1

_NC = 2
_NS = 16
_NW = _NC * _NS
_BPW = _N // _NW

_TCT = _TABLE // 128
_OCT = _N // 128

_AC = 4
_ABLK = _TCT // (_NW * _AC)
_AIT = _ABLK // 2

_C = 512
_NCH = _BPW // _C
_G = _C // 128
_HI = _C // 16
_OC = _C // 128

_P1 = -1640531535
_P2 = 805459861


def _worker_id():
    return lax.axis_index("s") * _NC + lax.axis_index("c")


def _quant(v):
    f = (v + 1.0) * 64.0
    f = jnp.minimum(jnp.maximum(f, 0.0), 127.0)
    return f.astype(jnp.int32)


def _table_transpose_kernel(raw_hbm, lin_hbm,
                            lo_a, hi_a, dst_a, lo_b, hi_b, dst_b,
                            s_in_a, s_in_b, s_out_a, s_out_b):
    base_ct = _worker_id() * (_AC * _ABLK)
    lanes16 = lax.iota(jnp.int32, 16) * _D

    def fire_in(blk, lo_v, hi_v, sem):
        c0 = (base_ct + blk * _AC) * 1024
        pltpu.async_copy(raw_hbm.at[pl.ds(c0, _AC * 1024)], lo_v, sem)
        pltpu.async_copy(
            raw_hbm.at[pl.ds(_TCT * 1024 + c0, _AC * 1024)], hi_v, sem)

    def drain_in(lo_v, hi_v, sem):
        pltpu.make_async_copy(raw_hbm.at[pl.ds(0, _AC * 1024)], lo_v,
                              sem).wait()
        pltpu.make_async_copy(raw_hbm.at[pl.ds(0, _AC * 1024)], hi_v,
                              sem).wait()

    def shuffle(lo_v, hi_v, dst_v):
        for c in range(_AC):
            for e in range(8):
                src = c * 1024 + e * 128
                for k in range(8):
                    di = lanes16 + (c * 2048 + k * 256 + e)
                    plsc.store_scatter(dst_v, [di],
                                       lo_v[pl.ds(src + k * 16, 16)])
                    plsc.store_scatter(dst_v, [di + 8],
                                       hi_v[pl.ds(src + k * 16, 16)])

    def fire_out(blk, dst_v, sem):
        o0 = (base_ct + blk * _AC) * 2048
        pltpu.async_copy(dst_v, lin_hbm.at[pl.ds(o0, _AC * 2048)], sem)

    def drain_out(dst_v, sem):
        pltpu.make_async_copy(raw_hbm.at[pl.ds(0, _AC * 2048)], dst_v,
                              sem).wait()

    fire_in(0, lo_a, hi_a, s_in_a)

    def it_body(i, carry):
        blk_a = 2 * i
        fire_in(blk_a + 1, lo_b, hi_b, s_in_b)
        drain_in(lo_a, hi_a, s_in_a)

        @pl.when(i > 0)
        def _():
            drain_out(dst_a, s_out_a)

        shuffle(lo_a, hi_a, dst_a)
        fire_out(blk_a, dst_a, s_out_a)

        @pl.when(i < _AIT - 1)
        def _():
            fire_in(blk_a + 2, lo_a, hi_a, s_in_a)

        drain_in(lo_b, hi_b, s_in_b)

        @pl.when(i > 0)
        def _():
            drain_out(dst_b, s_out_b)

        shuffle(lo_b, hi_b, dst_b)
        fire_out(blk_a + 1, dst_b, s_out_b)
        return carry

    lax.fori_loop(0, _AIT, it_body, 0)
    drain_out(dst_a, s_out_a)
    drain_out(dst_b, s_out_b)


def _gather_kernel(x_hbm, y_hbm, z_hbm, tbl_hbm, out_hbm,
                   x_v, y_v, z_v, idx_v, rows_v, t_a, t_b, sem, s_out):
    base = _worker_id() * _BPW
    li = lax.iota(jnp.int32, 16)
    pat16 = (li & 7) * 128 + (li >> 3) * 1024
    tbufs = [t_a, t_b]

    def chunk_body(c, carry):
        cb = base + c * _C
        pltpu.sync_copy(x_hbm.at[pl.ds(cb, _C)], x_v)
        pltpu.sync_copy(y_hbm.at[pl.ds(cb, _C)], y_v)
        pltpu.sync_copy(z_hbm.at[pl.ds(cb, _C)], z_v)
        for j in range(_HI):
            s = pl.ds(j * 16, 16)
            h = (_quant(x_v[s])
                 ^ (_quant(y_v[s]) * _P1)
                 ^ (_quant(z_v[s]) * _P2))
            idx_v[j // 8, pl.ds((j % 8) * 16, 16)] = h & _MASK
        copies = [
            pltpu.async_copy(
                tbl_hbm.at[idx_v.at[g]],
                rows_v.at[pl.ds(g * 128, 128)],
                sem,
            )
            for g in range(_G)
        ]
        for cp in copies:
            cp.wait()
        outs = []
        ct0 = cb // 128
        for oc in range(_OC):
            tb = tbufs[oc % 2]
            if oc >= 2:
                outs[2 * (oc - 2)].wait()
                outs[2 * (oc - 2) + 1].wait()
            for p in range(128):
                plsc.store_scatter(tb, [pat16 + p],
                                   rows_v[oc * 128 + p, :])
            o0 = (ct0 + oc) * 1024
            outs.append(pltpu.async_copy(
                tb.at[pl.ds(0, 1024)], out_hbm.at[pl.ds(o0, 1024)], s_out))
            outs.append(pltpu.async_copy(
                tb.at[pl.ds(1024, 1024)],
                out_hbm.at[pl.ds(_OCT * 1024 + o0, 1024)], s_out))
        for cp in outs[-4:]:
            cp.wait()
        return carry

    lax.fori_loop(0, _NCH, chunk_body, 0)


@jax.jit
def kernel(xyz, table):
    mesh = plsc.VectorSubcoreMesh(core_axis_name="c", subcore_axis_name="s")
    params = pltpu.CompilerParams(
        needs_layout_passes=False, use_tc_tiling_on_sc=False
    )

    transpose_table = functools.partial(
        pl.kernel,
        mesh=mesh,
        out_type=jax.ShapeDtypeStruct((_TABLE * _D,), jnp.float32),
        scratch_types=[
            pltpu.VMEM((_AC * 1024,), jnp.float32),
            pltpu.VMEM((_AC * 1024,), jnp.float32),
            pltpu.VMEM((_AC * 2048,), jnp.float32),
            pltpu.VMEM((_AC * 1024,), jnp.float32),
            pltpu.VMEM((_AC * 1024,), jnp.float32),
            pltpu.VMEM((_AC * 2048,), jnp.float32),
            pltpu.SemaphoreType.DMA,
            pltpu.SemaphoreType.DMA,
            pltpu.SemaphoreType.DMA,
            pltpu.SemaphoreType.DMA,
        ],
        compiler_params=params,
    )(_table_transpose_kernel)

    gather = functools.partial(
        pl.kernel,
        mesh=mesh,
        out_type=jax.ShapeDtypeStruct((_N * _D,), jnp.float32),
        scratch_types=[
            pltpu.VMEM((_C,), jnp.float32),
            pltpu.VMEM((_C,), jnp.float32),
            pltpu.VMEM((_C,), jnp.float32),
            pltpu.VMEM((_G, 128), jnp.int32),
            pltpu.VMEM((_C, _D), jnp.float32),
            pltpu.VMEM((2048,), jnp.float32),
            pltpu.VMEM((2048,), jnp.float32),
            pltpu.SemaphoreType.DMA,
            pltpu.SemaphoreType.DMA,
        ],
        compiler_params=params,
    )(_gather_kernel)

    x, y, z = xyz[:, 0], xyz[:, 1], xyz[:, 2]
    tbl_raw = (table.T.reshape(2, 8, _TCT, 128)
               .transpose(0, 2, 1, 3).reshape(_TABLE * _D))
    tbl_lin = transpose_table(tbl_raw)
    out_raw = gather(x, y, z, tbl_lin.reshape(_TABLE, _D))
    return (out_raw.reshape(2, _OCT, 8, 128)
            .transpose(0, 2, 1, 3).reshape(_D, _N).T)


# speedup vs baseline: 3.1353x; 1.0746x over previous
import functools

import jax
import jax.numpy as jnp
from jax import lax
from jax.experimental import pallas as pl
from jax.experimental.pallas import tpu as pltpu
from jax.experimental.pallas import tpu_sc as plsc

_N = 524288
_D = 16
_TABLE = 128 ** 3
_MASK = _TABLE - 1

_NC = 2
_NS = 16
_NW = _NC * _NS
_BPW = _N // _NW

_TCT = _TABLE // 128
_OCT = _N // 128

_AC = 4
_ASETS = 4
_ABLK = _TCT // (_NW * _AC)
_AIT = _ABLK // _ASETS

_C = 512
_NCH = _BPW // _C
_G = _C // 128
_HI = _C // 16
_OC = _C // 128

_P1 = -1640531535
_P2 = 805459861


def _worker_id():
    return lax.axis_index("s") * _NC + lax.axis_index("c")


def _quant(v):
    f = (v + 1.0) * 64.0
    f = jnp.minimum(jnp.maximum(f, 0.0), 127.0)
    return f.astype(jnp.int32)


def _table_transpose_kernel(raw_hbm, lin_hbm, *sc):
    lo = sc[0:4]
    hi = sc[4:8]
    dst = sc[8:12]
    s_in = sc[12:16]
    s_out = sc[16:20]
    base_ct = _worker_id() * (_AC * _ABLK)
    lanes16 = lax.iota(jnp.int32, 16) * _D

    def fire_in(blk, s):
        c0 = (base_ct + blk * _AC) * 1024
        pltpu.async_copy(raw_hbm.at[pl.ds(c0, _AC * 1024)], lo[s], s_in[s])
        pltpu.async_copy(
            raw_hbm.at[pl.ds(_TCT * 1024 + c0, _AC * 1024)], hi[s], s_in[s])

    def drain_in(s):
        pltpu.make_async_copy(raw_hbm.at[pl.ds(0, _AC * 1024)], lo[s],
                              s_in[s]).wait()
        pltpu.make_async_copy(raw_hbm.at[pl.ds(0, _AC * 1024)], hi[s],
                              s_in[s]).wait()

    def shuffle(s):
        for c in range(_AC):
            for e in range(8):
                src = c * 1024 + e * 128
                for k in range(8):
                    di = lanes16 + (c * 2048 + k * 256 + e)
                    plsc.store_scatter(dst[s], [di],
                                       lo[s][pl.ds(src + k * 16, 16)])
                    plsc.store_scatter(dst[s], [di + 8],
                                       hi[s][pl.ds(src + k * 16, 16)])

    def fire_out(blk, s):
        o0 = (base_ct + blk * _AC) * 2048
        pltpu.async_copy(dst[s], lin_hbm.at[pl.ds(o0, _AC * 2048)], s_out[s])

    def drain_out(s):
        pltpu.make_async_copy(raw_hbm.at[pl.ds(0, _AC * 2048)], dst[s],
                              s_out[s]).wait()

    for s in range(_ASETS - 1):
        fire_in(s, s)

    def it_body(i, carry):
        blk0 = _ASETS * i
        for s in range(_ASETS):
            blk = blk0 + s
            pre = blk + (_ASETS - 1)

            @pl.when(pre < _ABLK)
            def _():
                fire_in(pre, (s + _ASETS - 1) % _ASETS)

            drain_in(s)

            @pl.when(i > 0)
            def _():
                drain_out(s)

            shuffle(s)
            fire_out(blk, s)
        return carry

    lax.fori_loop(0, _AIT, it_body, 0)
    for s in range(_ASETS):
        drain_out(s)


def _gather_kernel(x_hbm, y_hbm, z_hbm, tbl_hbm, out_hbm,
                   x_v, y_v, z_v, idx0, idx1, rows0, rows1, t_a, t_b,
                   sg0, sg1, s_out):
    base = _worker_id() * _BPW
    li = lax.iota(jnp.int32, 16)
    pat16 = (li & 7) * 128 + (li >> 3) * 1024
    idx = [idx0, idx1]
    rows = [rows0, rows1]
    sg = [sg0, sg1]
    tbufs = [t_a, t_b]

    pltpu.sync_copy(x_hbm.at[pl.ds(base, _BPW)], x_v)
    pltpu.sync_copy(y_hbm.at[pl.ds(base, _BPW)], y_v)
    pltpu.sync_copy(z_hbm.at[pl.ds(base, _BPW)], z_v)

    def hash_fire(c, par):
        off = c * _C
        for j in range(_HI):
            s = pl.ds(off + j * 16, 16)
            h = (_quant(x_v[s])
                 ^ (_quant(y_v[s]) * _P1)
                 ^ (_quant(z_v[s]) * _P2))
            idx[par][j // 8, pl.ds((j % 8) * 16, 16)] = h & _MASK
        for g in range(_G):
            pltpu.async_copy(
                tbl_hbm.at[idx[par].at[g]],
                rows[par].at[pl.ds(g * 128, 128)],
                sg[par],
            )

    def finish(c, par):
        pltpu.make_async_copy(tbl_hbm.at[pl.ds(0, _C)], rows[par],
                              sg[par]).wait()
        outs = []
        ct0 = (base + c * _C) // 128
        for oc in range(_OC):
            tb = tbufs[oc % 2]
            if oc >= 2:
                outs[2 * (oc - 2)].wait()
                outs[2 * (oc - 2) + 1].wait()
            for p in range(128):
                plsc.store_scatter(tb, [pat16 + p],
                                   rows[par][oc * 128 + p, :])
            o0 = (ct0 + oc) * 1024
            outs.append(pltpu.async_copy(
                tb.at[pl.ds(0, 1024)], out_hbm.at[pl.ds(o0, 1024)], s_out))
            outs.append(pltpu.async_copy(
                tb.at[pl.ds(1024, 1024)],
                out_hbm.at[pl.ds(_OCT * 1024 + o0, 1024)], s_out))
        for cp in outs[-4:]:
            cp.wait()

    hash_fire(0, 0)

    def it_body(i, carry):
        a = 2 * i
        hash_fire(a + 1, 1)
        finish(a, 0)

        @pl.when(i < _NCH // 2 - 1)
        def _():
            hash_fire(a + 2, 0)

        finish(a + 1, 1)
        return carry

    lax.fori_loop(0, _NCH // 2, it_body, 0)


@jax.jit
def kernel(xyz, table):
    mesh = plsc.VectorSubcoreMesh(core_axis_name="c", subcore_axis_name="s")
    params = pltpu.CompilerParams(
        needs_layout_passes=False, use_tc_tiling_on_sc=False
    )

    transpose_table = functools.partial(
        pl.kernel,
        mesh=mesh,
        out_type=jax.ShapeDtypeStruct((_TABLE * _D,), jnp.float32),
        scratch_types=(
            [pltpu.VMEM((_AC * 1024,), jnp.float32)] * 8
            + [pltpu.VMEM((_AC * 2048,), jnp.float32)] * 4
            + [pltpu.SemaphoreType.DMA] * 8
        ),
        compiler_params=params,
    )(_table_transpose_kernel)

    gather = functools.partial(
        pl.kernel,
        mesh=mesh,
        out_type=jax.ShapeDtypeStruct((_N * _D,), jnp.float32),
        scratch_types=[
            pltpu.VMEM((_BPW,), jnp.float32),
            pltpu.VMEM((_BPW,), jnp.float32),
            pltpu.VMEM((_BPW,), jnp.float32),
            pltpu.VMEM((_G, 128), jnp.int32),
            pltpu.VMEM((_G, 128), jnp.int32),
            pltpu.VMEM((_C, _D), jnp.float32),
            pltpu.VMEM((_C, _D), jnp.float32),
            pltpu.VMEM((2048,), jnp.float32),
            pltpu.VMEM((2048,), jnp.float32),
            pltpu.SemaphoreType.DMA,
            pltpu.SemaphoreType.DMA,
            pltpu.SemaphoreType.DMA,
        ],
        compiler_params=params,
    )(_gather_kernel)

    x, y, z = xyz[:, 0], xyz[:, 1], xyz[:, 2]
    tbl_raw = (table.T.reshape(2, 8, _TCT, 128)
               .transpose(0, 2, 1, 3).reshape(_TABLE * _D))
    tbl_lin = transpose_table(tbl_raw)
    out_raw = gather(x, y, z, tbl_lin.reshape(_TABLE, _D))
    return (out_raw.reshape(2, _OCT, 8, 128)
            .transpose(0, 2, 1, 3).reshape(_D, _N).T)


# speedup vs baseline: 3.2548x; 1.0381x over previous
import functools

import jax
import jax.numpy as jnp
from jax import lax
from jax.experimental import pallas as pl
from jax.experimental.pallas import tpu as pltpu
from jax.experimental.pallas import tpu_sc as plsc

_N = 524288
_D = 16
_TABLE = 128 ** 3
_MASK = _TABLE - 1

_NC = 2
_NS = 16
_NW = _NC * _NS
_BPW = _N // _NW

_TCT = _TABLE // 128
_OCT = _N // 128

_AC = 8
_ASETS = 2
_ABLK = _TCT // (_NW * _AC)
_AIT = _ABLK // _ASETS

_C = 512
_NCH = _BPW // _C
_G = _C // 128
_HI = _C // 16
_OC = _C // 128

_P1 = -1640531535
_P2 = 805459861


def _worker_id():
    return lax.axis_index("s") * _NC + lax.axis_index("c")


def _quant(v):
    f = (v + 1.0) * 64.0
    f = jnp.minimum(jnp.maximum(f, 0.0), 127.0)
    return f.astype(jnp.int32)


def _table_transpose_kernel(raw_hbm, lin_hbm, *sc):
    lo = sc[0:_ASETS]
    hi = sc[_ASETS:2 * _ASETS]
    dst = sc[2 * _ASETS:3 * _ASETS]
    s_in = sc[3 * _ASETS:4 * _ASETS]
    s_out = sc[4 * _ASETS:5 * _ASETS]
    base_ct = _worker_id() * (_AC * _ABLK)
    lanes16 = lax.iota(jnp.int32, 16) * _D

    def fire_in(blk, s):
        c0 = (base_ct + blk * _AC) * 1024
        pltpu.async_copy(raw_hbm.at[pl.ds(c0, _AC * 1024)], lo[s], s_in[s])
        pltpu.async_copy(
            raw_hbm.at[pl.ds(_TCT * 1024 + c0, _AC * 1024)], hi[s], s_in[s])

    def drain_in(s):
        pltpu.make_async_copy(raw_hbm.at[pl.ds(0, _AC * 1024)], lo[s],
                              s_in[s]).wait()
        pltpu.make_async_copy(raw_hbm.at[pl.ds(0, _AC * 1024)], hi[s],
                              s_in[s]).wait()

    def shuffle(s):
        for c in range(_AC):
            for e in range(8):
                src = c * 1024 + e * 128
                for k in range(8):
                    di = lanes16 + (c * 2048 + k * 256 + e)
                    plsc.store_scatter(dst[s], [di],
                                       lo[s][pl.ds(src + k * 16, 16)])
                    plsc.store_scatter(dst[s], [di + 8],
                                       hi[s][pl.ds(src + k * 16, 16)])

    def fire_out(blk, s):
        o0 = (base_ct + blk * _AC) * 2048
        pltpu.async_copy(dst[s], lin_hbm.at[pl.ds(o0, _AC * 2048)], s_out[s])

    def drain_out(s):
        pltpu.make_async_copy(raw_hbm.at[pl.ds(0, _AC * 2048)], dst[s],
                              s_out[s]).wait()

    for s in range(_ASETS - 1):
        fire_in(s, s)

    def it_body(i, carry):
        blk0 = _ASETS * i
        for s in range(_ASETS):
            blk = blk0 + s
            pre = blk + (_ASETS - 1)

            @pl.when(pre < _ABLK)
            def _():
                fire_in(pre, (s + _ASETS - 1) % _ASETS)

            drain_in(s)

            @pl.when(i > 0)
            def _():
                drain_out(s)

            shuffle(s)
            fire_out(blk, s)
        return carry

    lax.fori_loop(0, _AIT, it_body, 0)
    for s in range(_ASETS):
        drain_out(s)


def _gather_kernel(x_hbm, y_hbm, z_hbm, tbl_hbm, out_hbm,
                   x_v, y_v, z_v, idx0, idx1, rows0, rows1, t_a, t_b,
                   sg0, sg1, s_out):
    base = _worker_id() * _BPW
    li = lax.iota(jnp.int32, 16)
    pat16 = (li & 7) * 128 + (li >> 3) * 1024
    idx = [idx0, idx1]
    rows = [rows0, rows1]
    sg = [sg0, sg1]
    tbufs = [t_a, t_b]

    pltpu.sync_copy(x_hbm.at[pl.ds(base, _BPW)], x_v)
    pltpu.sync_copy(y_hbm.at[pl.ds(base, _BPW)], y_v)
    pltpu.sync_copy(z_hbm.at[pl.ds(base, _BPW)], z_v)

    def hash_fire(c, par):
        off = c * _C
        for j in range(_HI):
            s = pl.ds(off + j * 16, 16)
            h = (_quant(x_v[s])
                 ^ (_quant(y_v[s]) * _P1)
                 ^ (_quant(z_v[s]) * _P2))
            idx[par][j // 8, pl.ds((j % 8) * 16, 16)] = h & _MASK
        for g in range(_G):
            pltpu.async_copy(
                tbl_hbm.at[idx[par].at[g]],
                rows[par].at[pl.ds(g * 128, 128)],
                sg[par],
            )

    def finish(c, par):
        pltpu.make_async_copy(tbl_hbm.at[pl.ds(0, _C)], rows[par],
                              sg[par]).wait()
        outs = []
        ct0 = (base + c * _C) // 128
        for oc in range(_OC):
            tb = tbufs[oc % 2]
            if oc >= 2:
                outs[2 * (oc - 2)].wait()
                outs[2 * (oc - 2) + 1].wait()
            for p in range(128):
                plsc.store_scatter(tb, [pat16 + p],
                                   rows[par][oc * 128 + p, :])
            o0 = (ct0 + oc) * 1024
            outs.append(pltpu.async_copy(
                tb.at[pl.ds(0, 1024)], out_hbm.at[pl.ds(o0, 1024)], s_out))
            outs.append(pltpu.async_copy(
                tb.at[pl.ds(1024, 1024)],
                out_hbm.at[pl.ds(_OCT * 1024 + o0, 1024)], s_out))
        for cp in outs[-4:]:
            cp.wait()

    hash_fire(0, 0)

    def it_body(i, carry):
        a = 2 * i
        hash_fire(a + 1, 1)
        finish(a, 0)

        @pl.when(i < _NCH // 2 - 1)
        def _():
            hash_fire(a + 2, 0)

        finish(a + 1, 1)
        return carry

    lax.fori_loop(0, _NCH // 2, it_body, 0)


@jax.jit
def kernel(xyz, table):
    mesh = plsc.VectorSubcoreMesh(core_axis_name="c", subcore_axis_name="s")
    params = pltpu.CompilerParams(
        needs_layout_passes=False, use_tc_tiling_on_sc=False
    )

    transpose_table = functools.partial(
        pl.kernel,
        mesh=mesh,
        out_type=jax.ShapeDtypeStruct((_TABLE * _D,), jnp.float32),
        scratch_types=(
            [pltpu.VMEM((_AC * 1024,), jnp.float32)] * (2 * _ASETS)
            + [pltpu.VMEM((_AC * 2048,), jnp.float32)] * _ASETS
            + [pltpu.SemaphoreType.DMA] * (2 * _ASETS)
        ),
        compiler_params=params,
    )(_table_transpose_kernel)

    gather = functools.partial(
        pl.kernel,
        mesh=mesh,
        out_type=jax.ShapeDtypeStruct((_N * _D,), jnp.float32),
        scratch_types=[
            pltpu.VMEM((_BPW,), jnp.float32),
            pltpu.VMEM((_BPW,), jnp.float32),
            pltpu.VMEM((_BPW,), jnp.float32),
            pltpu.VMEM((_G, 128), jnp.int32),
            pltpu.VMEM((_G, 128), jnp.int32),
            pltpu.VMEM((_C, _D), jnp.float32),
            pltpu.VMEM((_C, _D), jnp.float32),
            pltpu.VMEM((2048,), jnp.float32),
            pltpu.VMEM((2048,), jnp.float32),
            pltpu.SemaphoreType.DMA,
            pltpu.SemaphoreType.DMA,
            pltpu.SemaphoreType.DMA,
        ],
        compiler_params=params,
    )(_gather_kernel)

    x, y, z = xyz[:, 0], xyz[:, 1], xyz[:, 2]
    tbl_raw = (table.T.reshape(2, 8, _TCT, 128)
               .transpose(0, 2, 1, 3).reshape(_TABLE * _D))
    tbl_lin = transpose_table(tbl_raw)
    out_raw = gather(x, y, z, tbl_lin.reshape(_TABLE, _D))
    return (out_raw.reshape(2, _OCT, 8, 128)
            .transpose(0, 2, 1, 3).reshape(_D, _N).T)


# speedup vs baseline: 4.9217x; 1.5122x over previous
import functools

import jax
import jax.numpy as jnp
from jax import lax
from jax.experimental import pallas as pl
from jax.experimental.pallas import tpu as pltpu
from jax.experimental.pallas import tpu_sc as plsc

_N = 524288
_D = 16
_TABLE = 128 ** 3
_MASK = _TABLE - 1

_NC = 2
_NS = 16
_NW = _NC * _NS
_BPW = _N // _NW

_TCT = _TABLE // 128
_OCT = _N // 128

_C = 512
_NCH = _BPW // _C
_G = _C // 128
_HI = _C // 16
_OC = _C // 128

_P1 = -1640531535
_P2 = 805459861


def _worker_id():
    return lax.axis_index("s") * _NC + lax.axis_index("c")


def _quant(v):
    f = (v + 1.0) * 64.0
    f = jnp.minimum(jnp.maximum(f, 0.0), 127.0)
    return f.astype(jnp.int32)


_EOFF = [(e >> 3) * (_TCT * 1024) + (e & 7) * 128 for e in range(_D)]


def _gather_kernel(x_hbm, y_hbm, z_hbm, raw_hbm, out_hbm,
                   x_v, y_v, z_v, idx0, idx1, dst0, dst1,
                   sg0, sg1, so0, so1):
    base = _worker_id() * _BPW
    idx = [idx0, idx1]
    dst = [dst0, dst1]
    sg = [sg0, sg1]
    so = [so0, so1]

    pltpu.sync_copy(x_hbm.at[pl.ds(base, _BPW)], x_v)
    pltpu.sync_copy(y_hbm.at[pl.ds(base, _BPW)], y_v)
    pltpu.sync_copy(z_hbm.at[pl.ds(base, _BPW)], z_v)

    def drain(buf_par, sem, n):
        for _ in range(n):
            pltpu.make_async_copy(out_hbm.at[pl.ds(0, 16), :],
                                  buf_par.at[:, pl.ds(0, 128)], sem).wait()

    def hash_fire(c, par, first):
        off = c * _C
        for j in range(_HI):
            s = pl.ds(off + j * 16, 16)
            h = (_quant(x_v[s])
                 ^ (_quant(y_v[s]) * _P1)
                 ^ (_quant(z_v[s]) * _P2))
            h = h & _MASK
            a0 = ((h >> 7) * 1024) + (h & 127)
            for e in range(_D):
                idx[par][e, j // 8, pl.ds((j % 8) * 16, 16)] = a0 + _EOFF[e]
        if not first:
            drain(dst[par], so[par], 4)
        for e in range(_D):
            for g in range(_G):
                pltpu.async_copy(
                    raw_hbm.at[idx[par].at[e, g]],
                    dst[par].at[e, pl.ds(g * 128, 128)],
                    sg[par],
                )

    def finish(c, par):
        drain(dst[par], sg[par], 4)
        ct0 = (base + c * _C) // 128
        for r in range(2):
            for oc in range(_OC):
                row0 = (r * _OCT + ct0 + oc) * 8
                pltpu.async_copy(
                    dst[par].at[pl.ds(r * 8, 8), pl.ds(oc * 128, 128)],
                    out_hbm.at[pl.ds(row0, 8), :],
                    so[par],
                )

    hash_fire(0, 0, True)

    def it_body(i, carry):
        a = 2 * i

        @pl.when(i > 0)
        def _():
            hash_fire(a + 1, 1, False)

        @pl.when(i == 0)
        def _():
            hash_fire(a + 1, 1, True)

        finish(a, 0)

        @pl.when(i < _NCH // 2 - 1)
        def _():
            hash_fire(a + 2, 0, False)

        finish(a + 1, 1)
        return carry

    lax.fori_loop(0, _NCH // 2, it_body, 0)
    drain(dst[0], so[0], 4)
    drain(dst[1], so[1], 4)


@jax.jit
def kernel(xyz, table):
    mesh = plsc.VectorSubcoreMesh(core_axis_name="c", subcore_axis_name="s")
    params = pltpu.CompilerParams(
        needs_layout_passes=False, use_tc_tiling_on_sc=False
    )

    gather = functools.partial(
        pl.kernel,
        mesh=mesh,
        out_type=jax.ShapeDtypeStruct((_OCT * 16, 128), jnp.float32),
        scratch_types=[
            pltpu.VMEM((_BPW,), jnp.float32),
            pltpu.VMEM((_BPW,), jnp.float32),
            pltpu.VMEM((_BPW,), jnp.float32),
            pltpu.VMEM((_D, _G, 128), jnp.int32),
            pltpu.VMEM((_D, _G, 128), jnp.int32),
            pltpu.VMEM((_D, _C), jnp.float32),
            pltpu.VMEM((_D, _C), jnp.float32),
            pltpu.SemaphoreType.DMA,
            pltpu.SemaphoreType.DMA,
            pltpu.SemaphoreType.DMA,
            pltpu.SemaphoreType.DMA,
        ],
        compiler_params=params,
    )(_gather_kernel)

    x, y, z = xyz[:, 0], xyz[:, 1], xyz[:, 2]
    tbl_raw = (table.T.reshape(2, 8, _TCT, 128)
               .transpose(0, 2, 1, 3).reshape(_TABLE * _D))
    out2 = gather(x, y, z, tbl_raw)
    return (out2.reshape(2, _OCT, 8, 128)
            .transpose(0, 2, 1, 3).reshape(_D, _N).T)
